# baseline (speedup 1.0000x reference)
import jax
import jax.numpy as jnp
from jax.experimental import pallas as pl

_EPS = 1e-7


def _layernorm(x, g, b):
    mu = jnp.mean(x, axis=-1, keepdims=True)
    var = jnp.var(x, axis=-1, keepdims=True)
    return (x - mu) / jnp.sqrt(var + 1e-5) * g + b


def _genconv(x, src, dst, ea, t, W1, b1, g1, bt1, W2, b2):
    n = x.shape[0]
    msg = jax.nn.relu(x[src] + ea) + _EPS
    mt = msg * t
    mmax = jax.ops.segment_max(mt, dst, num_segments=n)
    mmax = jnp.where(jnp.isfinite(mmax), mmax, 0.0)
    ex = jnp.exp(mt - mmax[dst])
    denom = jax.ops.segment_sum(ex, dst, num_segments=n)
    alpha = ex / (denom[dst] + 1e-16)
    aggr = jax.ops.segment_sum(msg * alpha, dst, num_segments=n)
    out = aggr + x
    h = out @ W1 + b1
    h = jax.nn.relu(_layernorm(h, g1, bt1))
    return h @ W2 + b2


def kernel(x, edge_index, edge_attr, node_W, node_b, edge_W, edge_b, ts, W1, b1, mln_g, mln_b, W2, b2, norm_g, norm_b, lin_W, lin_b):
    L = W1.shape[0]
    src, dst = edge_index[0], edge_index[1]
    h = x @ node_W + node_b
    ea = edge_attr @ edge_W + edge_b
    h = _genconv(h, src, dst, ea, ts[0], W1[0], b1[0], mln_g[0], mln_b[0], W2[0], b2[0])
    for i in range(1, L):
        r = jax.nn.relu(_layernorm(h, norm_g[i], norm_b[i]))
        r = _genconv(r, src, dst, ea, ts[i], W1[i], b1[i], mln_g[i], mln_b[i], W2[i], b2[i])
        h = h + r
    h = jax.nn.relu(_layernorm(h, norm_g[0], norm_b[0]))
    return h @ lin_W + lin_b


# speedup vs baseline: 2.5017x; 2.5017x over previous
import functools

import jax
import jax.numpy as jnp
from jax import lax
from jax.experimental import pallas as pl
from jax.experimental.pallas import tpu as pltpu
from jax.experimental.pallas import tpu_sc as plsc

_EPS = 1e-7

_N = 50000
_NP = 50048
_E = 800000
_H = 64
_GW = 16
_NG = _H // _GW
_K = 128
_NCHUNK = _E // _K
_NSUB = 16
_NJ = -(-_NCHUNK // _NSUB)
_RPS = _NP // _NSUB
_ZROWS = 184
_NZ = _RPS // _ZROWS


def _sc_aggregate(r4, src, dst, ea4, caps):
    mesh = plsc.VectorSubcoreMesh(core_axis_name="c", subcore_axis_name="s")

    @functools.partial(
        pl.kernel,
        mesh=mesh,
        compiler_params=pltpu.CompilerParams(use_tc_tiling_on_sc=False),
        out_type=jax.ShapeDtypeStruct((_NG * _NP, _GW), jnp.float32),
        scratch_types=[
            pltpu.VMEM_SHARED((_NP, 2 * _GW), jnp.float32),
            pltpu.VMEM((_K,), jnp.int32),
            pltpu.VMEM((_K,), jnp.int32),
            pltpu.VMEM((_K, _GW), jnp.float32),
            pltpu.VMEM((_K, _GW), jnp.float32),
            pltpu.VMEM((_K, 2 * _GW), jnp.float32),
            pltpu.VMEM((_ZROWS, 2 * _GW), jnp.float32),
            pltpu.VMEM((_ZROWS, 2 * _GW), jnp.float32),
            pltpu.VMEM((_ZROWS, _GW), jnp.float32),
            pltpu.VMEM((_GW,), jnp.float32),
            pltpu.VMEM((_GW,), jnp.float32),
            pltpu.SemaphoreType.DMA,
        ],
    )
    def kern(r4_hbm, src_hbm, dst_hbm, ea4_hbm, caps_hbm, out_hbm,
             acc, idx_v, dst_v, ea_v, g_v, o_v, zbuf, fbuf, obuf,
             cap_v, t_v, sem):
        c = lax.axis_index("c")
        s = lax.axis_index("s")

        @pl.loop(0, _ZROWS)
        def _zinit(i):
            zbuf[i, pl.ds(0, _GW)] = jnp.zeros((_GW,), jnp.float32)
            zbuf[i, pl.ds(_GW, _GW)] = jnp.zeros((_GW,), jnp.float32)

        pltpu.sync_copy(caps_hbm.at[pl.ds(_H, _GW)], t_v)

        for p in range(2):
            g = p * 2 + c
            gN = g * _NP
            gE = g * _E
            pltpu.sync_copy(caps_hbm.at[pl.ds(g * _GW, _GW)], cap_v)

            @pl.loop(0, _NZ)
            def _zero(zi):
                pltpu.sync_copy(
                    zbuf, acc.at[pl.ds(s * _RPS + zi * _ZROWS, _ZROWS)])

            plsc.subcore_barrier()

            @pl.loop(0, _NJ)
            def _edges(j):
                ci = j * _NSUB + s

                @pl.when(ci < _NCHUNK)
                def _():
                    base = ci * _K
                    pltpu.sync_copy(src_hbm.at[pl.ds(base, _K)], idx_v)
                    pltpu.sync_copy(dst_hbm.at[pl.ds(base, _K)], dst_v)
                    pltpu.sync_copy(ea4_hbm.at[pl.ds(gE + base, _K)], ea_v)

                    @pl.loop(0, _K, step=_GW)
                    def _off(q):
                        idx_v[pl.ds(q, _GW)] = idx_v[pl.ds(q, _GW)] + gN

                    pltpu.async_copy(r4_hbm.at[idx_v], g_v, sem).wait()

                    tv = t_v[...]
                    capv = cap_v[...]

                    @pl.loop(0, _K)
                    def _compute(e):
                        m = jnp.maximum(g_v[e, :] + ea_v[e, :], 0.0) + _EPS
                        ex = jnp.exp(m * tv - capv)
                        o_v[e, pl.ds(0, _GW)] = ex
                        o_v[e, pl.ds(_GW, _GW)] = m * ex

                    pltpu.sync_copy(o_v, acc.at[dst_v], add=True)

            plsc.subcore_barrier()

            @pl.loop(0, _NZ)
            def _final(zi):
                rbase = s * _RPS + zi * _ZROWS
                pltpu.sync_copy(acc.at[pl.ds(rbase, _ZROWS)], fbuf)

                @pl.loop(0, _ZROWS)
                def _div(i):
                    d = fbuf[i, pl.ds(0, _GW)]
                    nm = fbuf[i, pl.ds(_GW, _GW)]
                    obuf[i, :] = jnp.where(d > 0.0, nm / d, 0.0)

                pltpu.sync_copy(obuf, out_hbm.at[pl.ds(gN + rbase, _ZROWS)])

    return kern(r4, src, dst, ea4, caps)


def _layernorm(x, g, b):
    mu = jnp.mean(x, axis=-1, keepdims=True)
    var = jnp.var(x, axis=-1, keepdims=True)
    return (x - mu) / jnp.sqrt(var + 1e-5) * g + b


def _to_groups(a, pad_to=None):
    r = a.shape[0]
    if pad_to is not None and pad_to != r:
        a = jnp.pad(a, ((0, pad_to - r), (0, 0)))
        r = pad_to
    return a.reshape(r, _NG, _GW).transpose(1, 0, 2).reshape(_NG * r, _GW)


def _from_groups(a):
    return a.reshape(_NG, _NP, _GW).transpose(1, 0, 2)[: _N].reshape(_N, _H)


def _build_caps(r, maxea, t):
    capmsg = jax.nn.relu(jnp.max(r, axis=0) + maxea) + _EPS
    capmt = jnp.maximum(t * capmsg, t * _EPS)
    return jnp.concatenate([capmt, jnp.full((_GW,), t, jnp.float32)])


def kernel(x, edge_index, edge_attr, node_W, node_b, edge_W, edge_b, ts,
           W1, b1, mln_g, mln_b, W2, b2, norm_g, norm_b, lin_W, lin_b):
    L = W1.shape[0]
    src, dst = edge_index[0], edge_index[1]
    h0 = x @ node_W + node_b
    ea = edge_attr @ edge_W + edge_b
    maxea = jnp.max(ea, axis=0)
    ea4 = _to_groups(ea)

    r = h0
    h = None
    for i in range(L):
        caps = _build_caps(r, maxea, ts[i])
        aggr = _from_groups(_sc_aggregate(_to_groups(r, _NP), src, dst, ea4, caps))
        out = aggr + r
        hmid = jax.nn.relu(_layernorm(out @ W1[i] + b1[i], mln_g[i], mln_b[i]))
        y = hmid @ W2[i] + b2[i]
        h = y if i == 0 else h + y
        if i + 1 < L:
            r = jax.nn.relu(_layernorm(h, norm_g[i + 1], norm_b[i + 1]))
    return jax.nn.relu(_layernorm(h, norm_g[0], norm_b[0])) @ lin_W + lin_b


# speedup vs baseline: 3.8690x; 1.5466x over previous
import functools

import jax
import jax.numpy as jnp
from jax import lax
from jax.experimental import pallas as pl
from jax.experimental.pallas import tpu as pltpu
from jax.experimental.pallas import tpu_sc as plsc

_EPS = 1e-7

_N = 50000
_NP = 50048
_E = 800000
_EP = 829440
_H = 64
_GW = 16
_NG = _H // _GW
_CW = 80
_CPG = 2
_GSZ = _CW * _CPG
_NGRP = 324
_NRND = 108
_NSUB = 16
_SPS = _EP // _NSUB
_RPSUB = _SPS // _CW
_IRT = _EP // _CW
_RPS = _NP // _NSUB
_ZCH = 64
_NZF = 48
_ZREM = _RPS - _NZF * _ZCH


def _sc_aggregate(r4, idx_all, dst2d, ea4, caps):
    mesh = plsc.VectorSubcoreMesh(core_axis_name="c", subcore_axis_name="s")

    @functools.partial(
        pl.kernel,
        mesh=mesh,
        compiler_params=pltpu.CompilerParams(use_tc_tiling_on_sc=False),
        out_type=jax.ShapeDtypeStruct((_NG * _NP, _GW), jnp.float32),
        scratch_types=[
            pltpu.VMEM_SHARED((_NP, 2 * _GW), jnp.float32),
            pltpu.VMEM((3, _CPG, _CW), jnp.int32),
            pltpu.VMEM((3, _CPG, _CW), jnp.int32),
            pltpu.VMEM((3, _GSZ * _GW), jnp.float32),
            pltpu.VMEM((3, _GSZ, _GW), jnp.float32),
            pltpu.VMEM((2, _CW, 2 * _GW), jnp.float32),
            pltpu.VMEM((2, _ZCH, 2 * _GW), jnp.float32),
            pltpu.VMEM((2, _ZCH, _GW), jnp.float32),
            pltpu.VMEM((_GW,), jnp.float32),
            pltpu.VMEM((_GW,), jnp.float32),
            pltpu.SemaphoreType.DMA,
            pltpu.SemaphoreType.DMA,
            pltpu.SemaphoreType.DMA,
            pltpu.SemaphoreType.DMA,
            pltpu.SemaphoreType.DMA,
            pltpu.SemaphoreType.DMA,
            pltpu.SemaphoreType.DMA,
            pltpu.SemaphoreType.DMA,
            pltpu.SemaphoreType.DMA,
            pltpu.SemaphoreType.DMA,
            pltpu.SemaphoreType.DMA,
            pltpu.SemaphoreType.DMA,
            pltpu.SemaphoreType.DMA,
        ],
    )
    def kern(idx_hbm, dst_hbm, r4_hbm, ea4_hbm, caps_hbm, out_hbm,
             acc, idx_b, dst_b, ea_b, g_b, o_r, fi_r, fo_r,
             cap_v, t_v,
             sem_ld0, sem_ld1, sem_ld2, sem_g0, sem_g1, sem_g2,
             sem_sc0, sem_sc1, sem_z, sem_fi0, sem_fi1, sem_fo0, sem_fo1):
        c = lax.axis_index("c")
        s = lax.axis_index("s")
        sem_ld = (sem_ld0, sem_ld1, sem_ld2)
        sem_g = (sem_g0, sem_g1, sem_g2)
        sem_sc = (sem_sc0, sem_sc1)
        sem_fi = (sem_fi0, sem_fi1)
        sem_fo = (sem_fo0, sem_fo1)

        pltpu.sync_copy(caps_hbm.at[pl.ds(_H, _GW)], t_v)

        @pl.loop(0, 2)
        def _pass(p):
            g = p * 2 + c
            gi = g * _IRT
            ge = g * _EP
            gn = g * _NP
            pltpu.sync_copy(caps_hbm.at[pl.ds(g * _GW, _GW)], cap_v)

            def load_descs(sl, q):
                ri = gi + s * _RPSUB + q * _CPG
                rd = s * _RPSUB + q * _CPG
                eb = ge + s * _SPS + q * _GSZ
                return (
                    pltpu.make_async_copy(
                        idx_hbm.at[pl.ds(ri, _CPG)], idx_b.at[sl], sem_ld[sl]),
                    pltpu.make_async_copy(
                        dst_hbm.at[pl.ds(rd, _CPG)], dst_b.at[sl], sem_ld[sl]),
                    pltpu.make_async_copy(
                        ea4_hbm.at[pl.ds(eb * _GW, _GSZ * _GW)],
                        ea_b.at[sl], sem_ld[sl]),
                )

            def gather_desc(sl, j):
                return pltpu.make_async_copy(
                    r4_hbm.at[idx_b.at[sl].at[j]],
                    g_b.at[sl].at[pl.ds(j * _CW, _CW)], sem_g[sl])

            def scatter_desc(sl, j):
                return pltpu.make_async_copy(
                    o_r.at[j % 2], acc.at[dst_b.at[sl].at[j]], sem_sc[j % 2])

            def compute_subchunk(sl, j):
                tv = t_v[...]
                capv = cap_v[...]
                ring = j % 2

                @pl.loop(0, _CW, step=4)
                def _(e):
                    for u in range(4):
                        ee = e + u
                        ga = g_b[sl, j * _CW + ee, :]
                        eav = ea_b[sl, pl.ds((j * _CW + ee) * _GW, _GW)]
                        m = jnp.maximum(ga + eav, 0.0) + _EPS
                        ex = jnp.exp(m * tv - capv)
                        o_r[ring, ee, pl.ds(0, _GW)] = ex
                        o_r[ring, ee, pl.ds(_GW, _GW)] = m * ex

                pltpu.async_copy(o_r.at[ring], acc.at[dst_b.at[sl].at[j]],
                                 sem_sc[ring], add=True)

            @pl.loop(0, _ZCH)
            def _zinit(i):
                fi_r[0, i, pl.ds(0, _GW)] = jnp.zeros((_GW,), jnp.float32)
                fi_r[0, i, pl.ds(_GW, _GW)] = jnp.zeros((_GW,), jnp.float32)

            @pl.loop(0, _NZF)
            def _zissue(k):
                pltpu.async_copy(
                    fi_r.at[0], acc.at[pl.ds(s * _RPS + k * _ZCH, _ZCH)],
                    sem_z)

            pltpu.async_copy(
                fi_r.at[0].at[pl.ds(0, _ZREM)],
                acc.at[pl.ds(s * _RPS + _NZF * _ZCH, _ZREM)], sem_z)
            for d in load_descs(0, 0):
                d.start()

            @pl.loop(0, _NZF)
            def _zdrain(k):
                pltpu.make_async_copy(
                    fi_r.at[0], acc.at[pl.ds(s * _RPS + k * _ZCH, _ZCH)],
                    sem_z).wait()

            pltpu.make_async_copy(
                fi_r.at[0].at[pl.ds(0, _ZREM)],
                acc.at[pl.ds(s * _RPS + _NZF * _ZCH, _ZREM)], sem_z).wait()

            plsc.subcore_barrier()

            for d in load_descs(0, 0):
                d.wait()
            for j in range(_CPG):
                gather_desc(0, j).start()
            for d in load_descs(1, 1):
                d.start()

            @pl.loop(0, _NRND)
            def _round(m):
                for r in range(3):
                    q = m * 3 + r
                    sl = r
                    sln = (r + 1) % 3
                    slp = (r + 2) % 3

                    @pl.when(q < _NGRP - 1)
                    def _():
                        for d in load_descs(sln, q + 1):
                            d.wait()
                        for j in range(_CPG):
                            gather_desc(sln, j).start()

                    for j in range(_CPG):
                        gather_desc(sl, j).wait()

                    for j in range(2):
                        @pl.when(q >= 1)
                        def _(j=j):
                            scatter_desc(slp, _CPG - 2 + j).wait()
                        compute_subchunk(sl, j)

                    @pl.when(q < _NGRP - 2)
                    def _():
                        for d in load_descs(slp, q + 2):
                            d.start()

                    for j in range(2, _CPG):
                        scatter_desc(sl, j - 2).wait()
                        compute_subchunk(sl, j)

            scatter_desc(2, _CPG - 2).wait()
            scatter_desc(2, _CPG - 1).wait()
            plsc.subcore_barrier()

            def fin_in_desc(k, rows, par):
                return pltpu.make_async_copy(
                    acc.at[pl.ds(s * _RPS + k * _ZCH, rows)],
                    fi_r.at[par].at[pl.ds(0, rows)], sem_fi[par])

            def fin_out_desc(k, rows, par):
                return pltpu.make_async_copy(
                    fo_r.at[par].at[pl.ds(0, rows)],
                    out_hbm.at[pl.ds(gn + s * _RPS + k * _ZCH, rows)],
                    sem_fo[par])

            fin_in_desc(0, _ZCH, 0).start()

            @pl.loop(0, _NZF // 2)
            def _fin(mm):
                for par in range(2):
                    k = mm * 2 + par

                    fin_in_desc(k, _ZCH, par).wait()

                    @pl.when(k < _NZF - 1)
                    def _():
                        fin_in_desc(k + 1, _ZCH, 1 - par).start()

                    @pl.when(k >= 2)
                    def _():
                        fin_out_desc(k - 2, _ZCH, par).wait()

                    @pl.loop(0, _ZCH)
                    def _div(i, par=par):
                        d = fi_r[par, i, pl.ds(0, _GW)]
                        nm = fi_r[par, i, pl.ds(_GW, _GW)]
                        fo_r[par, i, :] = jnp.where(d > 0.0, nm / d, 0.0)

                    fin_out_desc(k, _ZCH, par).start()

            fin_out_desc(_NZF - 2, _ZCH, 0).wait()
            fin_out_desc(_NZF - 1, _ZCH, 1).wait()
            pltpu.sync_copy(acc.at[pl.ds(s * _RPS + _NZF * _ZCH, _ZREM)],
                            fi_r.at[0].at[pl.ds(0, _ZREM)])

            @pl.loop(0, _ZREM)
            def _divr(i):
                d = fi_r[0, i, pl.ds(0, _GW)]
                nm = fi_r[0, i, pl.ds(_GW, _GW)]
                fo_r[0, i, :] = jnp.where(d > 0.0, nm / d, 0.0)

            pltpu.sync_copy(fo_r.at[0].at[pl.ds(0, _ZREM)],
                            out_hbm.at[pl.ds(gn + s * _RPS + _NZF * _ZCH,
                                             _ZREM)])

    return kern(idx_all, dst2d, r4, ea4, caps)


def _layernorm(x, g, b):
    mu = jnp.mean(x, axis=-1, keepdims=True)
    var = jnp.var(x, axis=-1, keepdims=True)
    return (x - mu) / jnp.sqrt(var + 1e-5) * g + b


def _to_groups(a, pad_to=None):
    r = a.shape[0]
    if pad_to is not None and pad_to != r:
        a = jnp.pad(a, ((0, pad_to - r), (0, 0)))
        r = pad_to
    return a.reshape(r, _NG, _GW).transpose(1, 0, 2).reshape(_NG * r, _GW)


def _from_groups(a):
    return a.reshape(_NG, _NP, _GW).transpose(1, 0, 2)[: _N].reshape(_N, _H)


def _build_caps(r, maxea, t):
    capmsg = jax.nn.relu(jnp.max(r, axis=0) + maxea) + _EPS
    capmt = jnp.maximum(t * capmsg, t * _EPS)
    return jnp.concatenate([capmt, jnp.full((_GW,), t, jnp.float32)])


def kernel(x, edge_index, edge_attr, node_W, node_b, edge_W, edge_b, ts,
           W1, b1, mln_g, mln_b, W2, b2, norm_g, norm_b, lin_W, lin_b):
    L = W1.shape[0]
    src, dst = edge_index[0], edge_index[1]
    h0 = x @ node_W + node_b
    ea = edge_attr @ edge_W + edge_b
    maxea = jnp.max(ea, axis=0)

    pad = _EP - _E
    srcp = jnp.concatenate([src, jnp.zeros((pad,), jnp.int32)])
    dstp = jnp.concatenate([dst, jnp.full((pad,), _N, jnp.int32)])
    idx_all = (srcp[None, :]
               + (jnp.arange(_NG, dtype=jnp.int32) * _NP)[:, None]
               ).reshape(_NG * _IRT, _CW)
    dst2d = dstp.reshape(_IRT, _CW)
    ea4 = _to_groups(ea, _EP).reshape(-1)

    r = h0
    h = None
    for i in range(L):
        caps = _build_caps(r, maxea, ts[i])
        aggr = _from_groups(
            _sc_aggregate(_to_groups(r, _NP), idx_all, dst2d, ea4, caps))
        out = aggr + r
        hmid = jax.nn.relu(_layernorm(out @ W1[i] + b1[i], mln_g[i], mln_b[i]))
        y = hmid @ W2[i] + b2[i]
        h = y if i == 0 else h + y
        if i + 1 < L:
            r = jax.nn.relu(_layernorm(h, norm_g[i + 1], norm_b[i + 1]))
    return jax.nn.relu(_layernorm(h, norm_g[0], norm_b[0])) @ lin_W + lin_b


# speedup vs baseline: 8.6833x; 2.2443x over previous
import functools

import jax
import jax.numpy as jnp
from jax import lax
from jax.experimental import pallas as pl
from jax.experimental.pallas import tpu as pltpu
from jax.experimental.pallas import tpu_sc as plsc

_EPS = 1e-7

_N = 50000
_NP = 50048
_E = 800000
_EP = 829440
_H = 64
_GW = 16
_NG = _H // _GW
_CW = 80
_CPG = 2
_GSZ = _CW * _CPG
_NGRP = 324
_NRND = 108
_NSUB = 16
_SPS = _EP // _NSUB
_RPSUB = _SPS // _CW
_IRT = _EP // _CW
_RPS = _NP // _NSUB
_ZCH = 64
_NZF = 48
_ZREM = _RPS - _NZF * _ZCH


def _sc_aggregate(r4, idx_all, dst2d, ea4, caps):
    mesh = plsc.VectorSubcoreMesh(core_axis_name="c", subcore_axis_name="s")

    @functools.partial(
        pl.kernel,
        mesh=mesh,
        compiler_params=pltpu.CompilerParams(use_tc_tiling_on_sc=False),
        out_type=jax.ShapeDtypeStruct((_NG * _NP, _GW), jnp.float32),
        scratch_types=[
            pltpu.VMEM_SHARED((_NP, 2 * _GW), jnp.float32),
            pltpu.VMEM((3, _CPG, _CW), jnp.int32),
            pltpu.VMEM((3, _CPG, _CW), jnp.int32),
            pltpu.VMEM((3, _GSZ * _GW), jnp.float32),
            pltpu.VMEM((3, _GSZ, _GW), jnp.float32),
            pltpu.VMEM((2, _CW, 2 * _GW), jnp.float32),
            pltpu.VMEM((2, _ZCH, 2 * _GW), jnp.float32),
            pltpu.VMEM((2, _ZCH, _GW), jnp.float32),
            pltpu.VMEM((_GW,), jnp.float32),
            pltpu.VMEM((_GW,), jnp.float32),
            pltpu.SemaphoreType.DMA,
            pltpu.SemaphoreType.DMA,
            pltpu.SemaphoreType.DMA,
            pltpu.SemaphoreType.DMA,
            pltpu.SemaphoreType.DMA,
            pltpu.SemaphoreType.DMA,
            pltpu.SemaphoreType.DMA,
            pltpu.SemaphoreType.DMA,
            pltpu.SemaphoreType.DMA,
            pltpu.SemaphoreType.DMA,
            pltpu.SemaphoreType.DMA,
            pltpu.SemaphoreType.DMA,
            pltpu.SemaphoreType.DMA,
        ],
    )
    def kern(idx_hbm, dst_hbm, r4_hbm, ea4_hbm, caps_hbm, out_hbm,
             acc, idx_b, dst_b, ea_b, g_b, o_r, fi_r, fo_r,
             cap_v, t_v,
             sem_ld0, sem_ld1, sem_ld2, sem_g0, sem_g1, sem_g2,
             sem_sc0, sem_sc1, sem_z, sem_fi0, sem_fi1, sem_fo0, sem_fo1):
        c = lax.axis_index("c")
        s = lax.axis_index("s")
        sem_ld = (sem_ld0, sem_ld1, sem_ld2)
        sem_g = (sem_g0, sem_g1, sem_g2)
        sem_sc = (sem_sc0, sem_sc1)
        sem_fi = (sem_fi0, sem_fi1)
        sem_fo = (sem_fo0, sem_fo1)

        pltpu.sync_copy(caps_hbm.at[pl.ds(_H, _GW)], t_v)

        @pl.loop(0, 2)
        def _pass(p):
            g = p * 2 + c
            gi = g * _IRT
            ge = g * _EP
            gn = g * _NP
            pltpu.sync_copy(caps_hbm.at[pl.ds(g * _GW, _GW)], cap_v)

            def load_descs(sl, q):
                ri = gi + s * _RPSUB + q * _CPG
                rd = s * _RPSUB + q * _CPG
                eb = ge + s * _SPS + q * _GSZ
                return (
                    pltpu.make_async_copy(
                        idx_hbm.at[pl.ds(ri, _CPG)], idx_b.at[sl], sem_ld[sl]),
                    pltpu.make_async_copy(
                        dst_hbm.at[pl.ds(rd, _CPG)], dst_b.at[sl], sem_ld[sl]),
                    pltpu.make_async_copy(
                        ea4_hbm.at[pl.ds(eb * _GW, _GSZ * _GW)],
                        ea_b.at[sl], sem_ld[sl]),
                )

            def gather_desc(sl, j):
                return pltpu.make_async_copy(
                    r4_hbm.at[idx_b.at[sl].at[j]],
                    g_b.at[sl].at[pl.ds(j * _CW, _CW)], sem_g[sl])

            def scatter_desc(sl, j):
                return pltpu.make_async_copy(
                    o_r.at[j % 2], acc.at[dst_b.at[sl].at[j]], sem_sc[j % 2])

            def compute_subchunk(sl, j):
                tv = t_v[...]
                capv = cap_v[...]
                ring = j % 2

                @plsc.parallel_loop(0, _CW, step=1, unroll=8)
                def _(ee):
                    ga = g_b[sl, j * _CW + ee, :]
                    eav = ea_b[sl, pl.ds((j * _CW + ee) * _GW, _GW)]
                    m = jnp.maximum(ga + eav, 0.0) + _EPS
                    ex = jnp.exp(m * tv - capv)
                    o_r[ring, ee, pl.ds(0, _GW)] = ex
                    o_r[ring, ee, pl.ds(_GW, _GW)] = m * ex

                pltpu.async_copy(o_r.at[ring], acc.at[dst_b.at[sl].at[j]],
                                 sem_sc[ring], add=True)

            @pl.loop(0, _ZCH)
            def _zinit(i):
                fi_r[0, i, pl.ds(0, _GW)] = jnp.zeros((_GW,), jnp.float32)
                fi_r[0, i, pl.ds(_GW, _GW)] = jnp.zeros((_GW,), jnp.float32)

            @pl.loop(0, _NZF)
            def _zissue(k):
                pltpu.async_copy(
                    fi_r.at[0], acc.at[pl.ds(s * _RPS + k * _ZCH, _ZCH)],
                    sem_z)

            pltpu.async_copy(
                fi_r.at[0].at[pl.ds(0, _ZREM)],
                acc.at[pl.ds(s * _RPS + _NZF * _ZCH, _ZREM)], sem_z)
            for d in load_descs(0, 0):
                d.start()

            @pl.loop(0, _NZF)
            def _zdrain(k):
                pltpu.make_async_copy(
                    fi_r.at[0], acc.at[pl.ds(s * _RPS + k * _ZCH, _ZCH)],
                    sem_z).wait()

            pltpu.make_async_copy(
                fi_r.at[0].at[pl.ds(0, _ZREM)],
                acc.at[pl.ds(s * _RPS + _NZF * _ZCH, _ZREM)], sem_z).wait()

            plsc.subcore_barrier()

            for d in load_descs(0, 0):
                d.wait()
            for j in range(_CPG):
                gather_desc(0, j).start()
            for d in load_descs(1, 1):
                d.start()

            @pl.loop(0, _NRND)
            def _round(m):
                for r in range(3):
                    q = m * 3 + r
                    sl = r
                    sln = (r + 1) % 3
                    slp = (r + 2) % 3

                    @pl.when(q < _NGRP - 1)
                    def _():
                        for d in load_descs(sln, q + 1):
                            d.wait()
                        for j in range(_CPG):
                            gather_desc(sln, j).start()

                    for j in range(_CPG):
                        gather_desc(sl, j).wait()

                    for j in range(2):
                        @pl.when(q >= 1)
                        def _(j=j):
                            scatter_desc(slp, _CPG - 2 + j).wait()
                        compute_subchunk(sl, j)

                    @pl.when(q < _NGRP - 2)
                    def _():
                        for d in load_descs(slp, q + 2):
                            d.start()

                    for j in range(2, _CPG):
                        scatter_desc(sl, j - 2).wait()
                        compute_subchunk(sl, j)

            scatter_desc(2, _CPG - 2).wait()
            scatter_desc(2, _CPG - 1).wait()
            plsc.subcore_barrier()

            def fin_in_desc(k, rows, par):
                return pltpu.make_async_copy(
                    acc.at[pl.ds(s * _RPS + k * _ZCH, rows)],
                    fi_r.at[par].at[pl.ds(0, rows)], sem_fi[par])

            def fin_out_desc(k, rows, par):
                return pltpu.make_async_copy(
                    fo_r.at[par].at[pl.ds(0, rows)],
                    out_hbm.at[pl.ds(gn + s * _RPS + k * _ZCH, rows)],
                    sem_fo[par])

            fin_in_desc(0, _ZCH, 0).start()

            @pl.loop(0, _NZF // 2)
            def _fin(mm):
                for par in range(2):
                    k = mm * 2 + par

                    fin_in_desc(k, _ZCH, par).wait()

                    @pl.when(k < _NZF - 1)
                    def _():
                        fin_in_desc(k + 1, _ZCH, 1 - par).start()

                    @pl.when(k >= 2)
                    def _():
                        fin_out_desc(k - 2, _ZCH, par).wait()

                    @plsc.parallel_loop(0, _ZCH, step=1, unroll=4)
                    def _div(i, par=par):
                        d = fi_r[par, i, pl.ds(0, _GW)]
                        nm = fi_r[par, i, pl.ds(_GW, _GW)]
                        fo_r[par, i, :] = jnp.where(d > 0.0, nm / d, 0.0)

                    fin_out_desc(k, _ZCH, par).start()

            fin_out_desc(_NZF - 2, _ZCH, 0).wait()
            fin_out_desc(_NZF - 1, _ZCH, 1).wait()
            pltpu.sync_copy(acc.at[pl.ds(s * _RPS + _NZF * _ZCH, _ZREM)],
                            fi_r.at[0].at[pl.ds(0, _ZREM)])

            @plsc.parallel_loop(0, _ZREM, step=1, unroll=4)
            def _divr(i):
                d = fi_r[0, i, pl.ds(0, _GW)]
                nm = fi_r[0, i, pl.ds(_GW, _GW)]
                fo_r[0, i, :] = jnp.where(d > 0.0, nm / d, 0.0)

            pltpu.sync_copy(fo_r.at[0].at[pl.ds(0, _ZREM)],
                            out_hbm.at[pl.ds(gn + s * _RPS + _NZF * _ZCH,
                                             _ZREM)])

    return kern(idx_all, dst2d, r4, ea4, caps)


def _layernorm(x, g, b):
    mu = jnp.mean(x, axis=-1, keepdims=True)
    var = jnp.var(x, axis=-1, keepdims=True)
    return (x - mu) / jnp.sqrt(var + 1e-5) * g + b


def _to_groups(a, pad_to=None):
    r = a.shape[0]
    if pad_to is not None and pad_to != r:
        a = jnp.pad(a, ((0, pad_to - r), (0, 0)))
        r = pad_to
    return a.reshape(r, _NG, _GW).transpose(1, 0, 2).reshape(_NG * r, _GW)


def _from_groups(a):
    return a.reshape(_NG, _NP, _GW).transpose(1, 0, 2)[: _N].reshape(_N, _H)


def _build_caps(r, maxea, t):
    capmsg = jax.nn.relu(jnp.max(r, axis=0) + maxea) + _EPS
    capmt = jnp.maximum(t * capmsg, t * _EPS)
    return jnp.concatenate([capmt, jnp.full((_GW,), t, jnp.float32)])


def kernel(x, edge_index, edge_attr, node_W, node_b, edge_W, edge_b, ts,
           W1, b1, mln_g, mln_b, W2, b2, norm_g, norm_b, lin_W, lin_b):
    L = W1.shape[0]
    src, dst = edge_index[0], edge_index[1]
    h0 = x @ node_W + node_b
    ea = edge_attr @ edge_W + edge_b
    maxea = jnp.max(ea, axis=0)

    pad = _EP - _E
    srcp = jnp.concatenate([src, jnp.zeros((pad,), jnp.int32)])
    dstp = jnp.concatenate([dst, jnp.full((pad,), _N, jnp.int32)])
    idx_all = (srcp[None, :]
               + (jnp.arange(_NG, dtype=jnp.int32) * _NP)[:, None]
               ).reshape(_NG * _IRT, _CW)
    dst2d = dstp.reshape(_IRT, _CW)
    ea4 = _to_groups(ea, _EP).reshape(-1)

    r = h0
    h = None
    for i in range(L):
        caps = _build_caps(r, maxea, ts[i])
        aggr = _from_groups(
            _sc_aggregate(_to_groups(r, _NP), idx_all, dst2d, ea4, caps))
        out = aggr + r
        hmid = jax.nn.relu(_layernorm(out @ W1[i] + b1[i], mln_g[i], mln_b[i]))
        y = hmid @ W2[i] + b2[i]
        h = y if i == 0 else h + y
        if i + 1 < L:
            r = jax.nn.relu(_layernorm(h, norm_g[i + 1], norm_b[i + 1]))
    return jax.nn.relu(_layernorm(h, norm_g[0], norm_b[0])) @ lin_W + lin_b
